# baseline (speedup 1.0000x reference)
import jax
import jax.numpy as jnp
from jax import lax
from jax.experimental import pallas as pl
from jax.experimental.pallas import tpu as pltpu
from jax.experimental.pallas import tpu_sc as plsc

_B = 16384
_D = 32
_NCAT = 8
_NCONT = 4
_OUTW = _NCONT + _NCAT * _D

_NW = 32
_CHUNK = 256
_NCHUNK = _B // (_NW * _CHUNK)
_Q = _CHUNK // 128


def _body(*refs):
    cats = refs[0:_NCAT]
    tables = refs[_NCAT:2 * _NCAT]
    out = refs[2 * _NCAT]
    idxs = refs[2 * _NCAT + 1:3 * _NCAT + 1]
    rows_v = refs[3 * _NCAT + 1]
    gsem = refs[3 * _NCAT + 2]
    wsem = refs[3 * _NCAT + 3]

    wid = lax.axis_index("s") * 2 + lax.axis_index("c")
    for h in range(_NCHUNK):
        rowbase = wid * (_CHUNK * _NCHUNK) + h * _CHUNK
        for j in range(_NCAT):
            pltpu.sync_copy(cats[j].at[pl.ds(rowbase, _CHUNK)], idxs[j])
        gh = []
        for j in range(_NCAT):
            for q in range(_Q):
                gh.append(pltpu.async_copy(
                    tables[j].at[idxs[j].at[pl.ds(q * 128, 128)]],
                    rows_v.at[j, pl.ds(q * 128, 128)],
                    gsem))
        wh = []
        for j in range(_NCAT):
            gh[2 * j].wait()
            gh[2 * j + 1].wait()
            wh.append(pltpu.async_copy(
                rows_v.at[j],
                out.at[pl.ds(rowbase, _CHUNK), j],
                wsem))
        for hnd in wh:
            hnd.wait()


def kernel(cat_0, cat_1, cat_2, cat_3, cat_4, cat_5, cat_6, cat_7,
           table_0, table_1, table_2, table_3, table_4, table_5, table_6, table_7,
           cont_0, cont_1, cont_2, cont_3):
    cats = [c.astype(jnp.int32).reshape(_B)
            for c in (cat_0, cat_1, cat_2, cat_3, cat_4, cat_5, cat_6, cat_7)]
    tables = (table_0, table_1, table_2, table_3, table_4, table_5, table_6, table_7)

    mesh = plsc.VectorSubcoreMesh(core_axis_name="c", subcore_axis_name="s")
    k = pl.kernel(
        _body,
        mesh=mesh,
        compiler_params=pltpu.CompilerParams(use_tc_tiling_on_sc=False),
        out_type=jax.ShapeDtypeStruct((_B, _NCAT, _D), jnp.float32),
        scratch_types=(
            [pltpu.VMEM((_CHUNK,), jnp.int32) for _ in range(_NCAT)]
            + [pltpu.VMEM((_NCAT, _CHUNK, _D), jnp.float32),
               pltpu.SemaphoreType.DMA,
               pltpu.SemaphoreType.DMA]
        ),
    )
    embs = k(*cats, *tables).reshape(_B, _NCAT * _D)
    cont = jnp.stack(
        [c.astype(jnp.float32) for c in (cont_0, cont_1, cont_2, cont_3)],
        axis=-1)
    return jnp.concatenate([cont, embs], axis=-1)


# speedup vs baseline: 1.0048x; 1.0048x over previous
import jax
import jax.numpy as jnp
from jax import lax
from jax.experimental import pallas as pl
from jax.experimental.pallas import tpu as pltpu
from jax.experimental.pallas import tpu_sc as plsc

_B = 16384
_D = 32
_NCAT = 8
_NCONT = 4
_OUTW = _NCONT + _NCAT * _D

_NW = 32
_CHUNK = 256
_NCHUNK = _B // (_NW * _CHUNK)
_Q = _CHUNK // 128


def _body(*refs):
    cats = refs[0]
    tables = refs[1:_NCAT + 1]
    out = refs[_NCAT + 1]
    idxs = refs[_NCAT + 2:2 * _NCAT + 2]
    rows_v = refs[2 * _NCAT + 2]
    gsem = refs[2 * _NCAT + 3]
    wsem = refs[2 * _NCAT + 4]

    wid = lax.axis_index("s") * 2 + lax.axis_index("c")
    for h in range(_NCHUNK):
        rowbase = wid * (_CHUNK * _NCHUNK) + h * _CHUNK
        for j in range(_NCAT):
            pltpu.sync_copy(cats.at[j, pl.ds(rowbase, _CHUNK)], idxs[j])
        gh = []
        for j in range(_NCAT):
            for q in range(_Q):
                gh.append(pltpu.async_copy(
                    tables[j].at[idxs[j].at[pl.ds(q * 128, 128)]],
                    rows_v.at[j, pl.ds(q * 128, 128)],
                    gsem))
        wh = []
        for j in range(_NCAT):
            gh[2 * j].wait()
            gh[2 * j + 1].wait()
            wh.append(pltpu.async_copy(
                rows_v.at[j],
                out.at[pl.ds(rowbase, _CHUNK), j],
                wsem))
        for hnd in wh:
            hnd.wait()


def kernel(cat_0, cat_1, cat_2, cat_3, cat_4, cat_5, cat_6, cat_7,
           table_0, table_1, table_2, table_3, table_4, table_5, table_6, table_7,
           cont_0, cont_1, cont_2, cont_3):
    cats = jnp.stack(
        [c.astype(jnp.int32).reshape(_B)
         for c in (cat_0, cat_1, cat_2, cat_3, cat_4, cat_5, cat_6, cat_7)],
        axis=0)
    tables = (table_0, table_1, table_2, table_3, table_4, table_5, table_6, table_7)

    mesh = plsc.VectorSubcoreMesh(core_axis_name="c", subcore_axis_name="s")
    k = pl.kernel(
        _body,
        mesh=mesh,
        compiler_params=pltpu.CompilerParams(use_tc_tiling_on_sc=False),
        out_type=jax.ShapeDtypeStruct((_B, _NCAT, _D), jnp.float32),
        scratch_types=(
            [pltpu.VMEM((_CHUNK,), jnp.int32) for _ in range(_NCAT)]
            + [pltpu.VMEM((_NCAT, _CHUNK, _D), jnp.float32),
               pltpu.SemaphoreType.DMA,
               pltpu.SemaphoreType.DMA]
        ),
    )
    embs = k(cats, *tables).reshape(_B, _NCAT * _D)
    cont = jnp.stack(
        [c.astype(jnp.float32) for c in (cont_0, cont_1, cont_2, cont_3)],
        axis=-1)
    return jnp.concatenate([cont, embs], axis=-1)
